# baseline (speedup 1.0000x reference)
import functools

import jax
import jax.numpy as jnp
from jax.experimental import pallas as pl
from jax.experimental.pallas import tpu as pltpu
from jax.experimental.pallas import tpu_sc as plsc

B = 16
HID = 2048
NH = 16
NOPE = 64
ROPE = 32
VD = 64
QD = NOPE + ROPE
E = 8
INTER = 1408
KV = 2048
EPS = 1e-06
SCALE = QD ** -0.5

HD = NH * NOPE
BLK = 256
NBLK = KV // BLK
CHUNK = 128
NC = INTER // CHUNK
E2 = E // 2

_F32 = jnp.float32
_DN = (((1,), (1,)), ((), ()))
_DN0 = (((1,), (0,)), ((), ()))


def _rms(x, w):
    return x * jax.lax.rsqrt(jnp.mean(x * x, axis=-1, keepdims=True) + EPS) * w


def _pre_body(x_ref, w_ref, wq_ref, wkv_ref, qn_ref, kc_ref, vc_ref):
    xn = _rms(x_ref[...], w_ref[...])
    wq = wq_ref[...].reshape(HD, HID)
    qn_ref[...] = jax.lax.dot_general(xn, wq, _DN,
                                      preferred_element_type=_F32) * SCALE
    kv = jax.lax.dot_general(xn, wkv_ref[...], _DN,
                             preferred_element_type=_F32)
    kc_ref[...] = kv[:, :HD]
    vc_ref[...] = kv[:, HD:2 * HD]


def _attn_body(sl_ref, q_ref, kc_ref, vc_ref, kvc_ref,
               o_ref, q3s, acc, m, l, kbuf, vbuf, sems):
    b = pl.program_id(0)
    j = pl.program_id(1)
    ncache = sl_ref[b] - 1
    nb = (ncache + BLK - 1) // BLK

    def _copy(i, slot, part, buf):
        return pltpu.make_async_copy(
            kvc_ref.at[part, b, pl.ds(i * BLK, BLK)],
            buf.at[slot], sems.at[part, slot])

    @pl.when(j == 0)
    def _init():
        q3 = q_ref[...].reshape(NH, NOPE)
        q3s[...] = q3
        kc3 = kc_ref[...].reshape(NH, NOPE)
        s_cur = jnp.sum(q3 * kc3, axis=1, keepdims=True)
        m[...] = s_cur
        l[...] = jnp.ones_like(s_cur)
        acc[...] = vc_ref[...].reshape(NH, NOPE)

        @pl.when(nb > 0)
        def _prologue():
            _copy(0, 0, 0, kbuf).start()
            _copy(0, 0, 1, vbuf).start()

    @pl.when(j < nb)
    def _block():
        slot = jax.lax.rem(j, 2)
        _copy(j, slot, 0, kbuf).wait()
        _copy(j, slot, 1, vbuf).wait()

        @pl.when(j + 1 < nb)
        def _prefetch():
            nslot = jax.lax.rem(j + 1, 2)
            _copy(j + 1, nslot, 0, kbuf).start()
            _copy(j + 1, nslot, 1, vbuf).start()

        kb3 = kbuf[slot]
        vb3 = vbuf[slot]
        prod3 = kb3 * q3s[...][None]
        s = jnp.sum(prod3, axis=2)
        pos = j * BLK + jax.lax.broadcasted_iota(jnp.int32, (BLK, NH), 0)
        s = jnp.where(pos < ncache, s, -1e30)
        bm = jnp.max(s, axis=0, keepdims=True)
        new_m = jnp.maximum(m[...].T, bm)
        new_mc = new_m.T
        corr = jnp.exp(m[...] - new_mc)
        p = jnp.exp(s - new_m)
        l[...] = l[...] * corr + jnp.sum(p, axis=0, keepdims=True).T
        m[...] = new_mc
        pv3 = jax.lax.broadcast_in_dim(p, (BLK, NH, NOPE), (0, 1))
        acc[...] = acc[...] * corr + jnp.sum(pv3 * vb3, axis=0)

    @pl.when(j == NBLK - 1)
    def _fin():
        o_ref[...] = (acc[...] / l[...]).reshape(1, NH, NOPE)


def _mid_body(ao_ref, x_ref, wo_ref, wn_ref, wg_ref, wse_ref, wsd_ref,
              h2_ref, wt_ref, base_ref):
    attn_res = jax.lax.dot_general(ao_ref[...], wo_ref[...], _DN,
                                   preferred_element_type=_F32)
    resid = x_ref[...] + attn_res
    h2 = _rms(resid, wn_ref[...])
    h2_ref[...] = h2
    wt_ref[...] = jax.lax.dot_general(wg_ref[...], h2, _DN,
                                      preferred_element_type=_F32)
    su = jax.lax.dot_general(h2, wse_ref[...], _DN,
                             preferred_element_type=_F32)
    sg = su[:, :INTER]
    uu = su[:, INTER:]
    act = sg * jax.nn.sigmoid(sg) * uu
    shared = jax.lax.dot_general(act, wsd_ref[...], _DN,
                                 preferred_element_type=_F32)
    base_ref[...] = resid + shared


def _route_sc_body(lg_hbm, out_hbm, lg_v, wt_v):
    cid = jax.lax.axis_index("c")
    sid = jax.lax.axis_index("s")

    @pl.when((cid == 0) & (sid == 0))
    def _():
        pltpu.sync_copy(lg_hbm, lg_v)
        rows = [lg_v[e, :] for e in range(E)]
        m1 = rows[0]
        for e in range(1, E):
            m1 = jnp.maximum(m1, rows[e])
        i1 = jnp.full((B,), E, jnp.int32)
        for e in range(E - 1, -1, -1):
            i1 = jnp.where(rows[e] == m1, jnp.int32(e), i1)
        neg = jnp.full((B,), -jnp.inf, jnp.float32)
        m2 = neg
        for e in range(E):
            m2 = jnp.maximum(m2, jnp.where(i1 == e, neg, rows[e]))
        i2 = jnp.full((B,), E, jnp.int32)
        for e in range(E - 1, -1, -1):
            v = jnp.where(i1 == e, neg, rows[e])
            i2 = jnp.where(v == m2, jnp.int32(e), i2)
        e2 = jnp.exp(m2 - m1)
        d = 1.0 + e2
        wa = 1.0 / d
        wb = e2 / d
        z = jnp.zeros((B,), jnp.float32)
        for e in range(E):
            wt_v[e, :] = jnp.where(i1 == e, wa, z) + jnp.where(i2 == e, wb, z)
        pltpu.sync_copy(wt_v, out_hbm)


def _route_sc(lgt):
    return pl.kernel(
        _route_sc_body,
        out_type=jax.ShapeDtypeStruct((E, B), jnp.float32),
        mesh=plsc.VectorSubcoreMesh(core_axis_name="c", subcore_axis_name="s"),
        scratch_types=[pltpu.VMEM((E, B), jnp.float32),
                       pltpu.VMEM((E, B), jnp.float32)],
    )(lgt)


def _moe_body(x_ref, wt_ref, w1g_ref, w1u_ref, w2_ref, o_ref):
    e = pl.program_id(1)
    c = pl.program_id(2)

    @pl.when((e == 0) & (c == 0))
    def _zero():
        o_ref[...] = jnp.zeros_like(o_ref)

    x = x_ref[...]
    g = jax.lax.dot_general(x, w1g_ref[...].reshape(CHUNK, HID), _DN,
                            preferred_element_type=_F32)
    u = jax.lax.dot_general(x, w1u_ref[...].reshape(CHUNK, HID), _DN,
                            preferred_element_type=_F32)
    wcol = wt_ref[...].reshape(B, 1)
    act = g * jax.nn.sigmoid(g) * u * wcol
    o_ref[...] += jax.lax.dot_general(
        act, w2_ref[...].reshape(HID, CHUNK), _DN,
        preferred_element_type=_F32)[None]


def kernel(hidden_states, positions, kv_cache, seq_lens, slot_mapping,
           ln1_w, ln2_w, Wq, Wkv, Wo, Wg, w1, w2, Wse, Wsd):
    x = hidden_states[:, 0, :]
    sl = jnp.maximum(seq_lens, 1).astype(jnp.int32)
    Wq3 = Wq.reshape(NH, QD, HID)

    q_nope, k_cur, v_cur = pl.pallas_call(
        _pre_body,
        grid=(1,),
        in_specs=[
            pl.BlockSpec((B, HID), lambda i: (0, 0)),
            pl.BlockSpec((1, HID), lambda i: (0, 0)),
            pl.BlockSpec((NH, NOPE, HID), lambda i: (0, 0, 0)),
            pl.BlockSpec((2 * HD, HID), lambda i: (0, 0)),
        ],
        out_specs=[pl.BlockSpec((B, HD), lambda i: (0, 0))] * 3,
        out_shape=[jax.ShapeDtypeStruct((B, HD), _F32),
                   jax.ShapeDtypeStruct((B, HD), _F32),
                   jax.ShapeDtypeStruct((B, HD), _F32)],
    )(x, ln1_w.reshape(1, HID), Wq3, Wkv)

    q_nope = q_nope.reshape(B, NH, NOPE)
    k_cur = k_cur.reshape(B, NH, NOPE)
    v_cur = v_cur.reshape(B, NH, NOPE)

    attn_out = pl.pallas_call(
        _attn_body,
        grid_spec=pltpu.PrefetchScalarGridSpec(
            num_scalar_prefetch=1,
            grid=(B, NBLK),
            in_specs=[
                pl.BlockSpec((1, NH, NOPE), lambda b, j, s: (b, 0, 0)),
                pl.BlockSpec((1, NH, NOPE), lambda b, j, s: (b, 0, 0)),
                pl.BlockSpec((1, NH, NOPE), lambda b, j, s: (b, 0, 0)),
                pl.BlockSpec(memory_space=pl.ANY),
            ],
            out_specs=pl.BlockSpec((1, NH, NOPE), lambda b, j, s: (b, 0, 0)),
            scratch_shapes=[pltpu.VMEM((NH, NOPE), _F32),
                            pltpu.VMEM((NH, NOPE), _F32),
                            pltpu.VMEM((NH, 1), _F32),
                            pltpu.VMEM((NH, 1), _F32),
                            pltpu.VMEM((2, BLK, NH, NOPE), _F32),
                            pltpu.VMEM((2, BLK, NH, NOPE), _F32),
                            pltpu.SemaphoreType.DMA((2, 2))],
        ),
        out_shape=jax.ShapeDtypeStruct((B, NH, NOPE), _F32),
        compiler_params=pltpu.CompilerParams(
            dimension_semantics=("arbitrary", "arbitrary")),
    )(sl, q_nope, k_cur, v_cur, kv_cache)

    h2, lgt, base = pl.pallas_call(
        _mid_body,
        out_shape=[jax.ShapeDtypeStruct((B, HID), _F32),
                   jax.ShapeDtypeStruct((E, B), _F32),
                   jax.ShapeDtypeStruct((B, HID), _F32)],
    )(attn_out.reshape(B, HD), x, Wo, ln2_w.reshape(1, HID), Wg, Wse, Wsd)

    wt3 = _route_sc(lgt)[:, :, None]

    moe = pl.pallas_call(
        _moe_body,
        grid=(2, E2, NC),
        in_specs=[
            pl.BlockSpec((B, HID), lambda p, e, c: (0, 0)),
            pl.BlockSpec((1, B, 1), lambda p, e, c: (p * E2 + e, 0, 0)),
            pl.BlockSpec((1, CHUNK, HID), lambda p, e, c: (p * E2 + e, c, 0)),
            pl.BlockSpec((1, CHUNK, HID),
                         lambda p, e, c: (p * E2 + e, NC + c, 0)),
            pl.BlockSpec((1, HID, CHUNK), lambda p, e, c: (p * E2 + e, 0, c)),
        ],
        out_specs=pl.BlockSpec((1, B, HID), lambda p, e, c: (p, 0, 0)),
        out_shape=jax.ShapeDtypeStruct((2, B, HID), _F32),
        compiler_params=pltpu.CompilerParams(
            dimension_semantics=("parallel", "arbitrary", "arbitrary")),
    )(h2, wt3, w1, w1, w2)

    out = base + moe[0] + moe[1]
    return out[:, None, :]


# speedup vs baseline: 1.5836x; 1.5836x over previous
import functools

import jax
import jax.numpy as jnp
from jax.experimental import pallas as pl
from jax.experimental.pallas import tpu as pltpu
from jax.experimental.pallas import tpu_sc as plsc

B = 16
HID = 2048
NH = 16
NOPE = 64
ROPE = 32
VD = 64
QD = NOPE + ROPE
E = 8
INTER = 1408
KV = 2048
EPS = 1e-06
SCALE = QD ** -0.5

HD = NH * NOPE
BLK = 256
NBLK = KV // BLK
CHUNK = 128
NC = INTER // CHUNK
E2 = E // 2

_F32 = jnp.float32
_DN = (((1,), (1,)), ((), ()))
_DN0 = (((1,), (0,)), ((), ()))


def _rms(x, w):
    return x * jax.lax.rsqrt(jnp.mean(x * x, axis=-1, keepdims=True) + EPS) * w


def _pre_body(x_ref, w_ref, wq_ref, wkv_ref, qn_ref, kc_ref, vc_ref):
    xn = _rms(x_ref[...], w_ref[...])
    wq = wq_ref[...].reshape(HD, HID)
    qn_ref[...] = jax.lax.dot_general(xn, wq, _DN,
                                      preferred_element_type=_F32) * SCALE
    kv = jax.lax.dot_general(xn, wkv_ref[...], _DN,
                             preferred_element_type=_F32)
    kc_ref[...] = kv[:, :HD]
    vc_ref[...] = kv[:, HD:2 * HD]


def _attn_body(sl_ref, q_ref, kc_ref, vc_ref, kb_ref, vb_ref,
               o_ref, q3s, acc, m, l):
    b = pl.program_id(0)
    j = pl.program_id(1)
    ncache = sl_ref[b] - 1
    nb = (ncache + BLK - 1) // BLK

    @pl.when(j == 0)
    def _init():
        q3 = q_ref[...].reshape(NH, NOPE)
        q3s[...] = q3
        kc3 = kc_ref[...].reshape(NH, NOPE)
        s_cur = jnp.sum(q3 * kc3, axis=1, keepdims=True)
        m[...] = s_cur
        l[...] = jnp.ones_like(s_cur)
        acc[...] = vc_ref[...].reshape(NH, NOPE)

    @pl.when(j < nb)
    def _block():
        kb3 = kb_ref[...].reshape(BLK, NH, NOPE)
        vb3 = vb_ref[...].reshape(BLK, NH, NOPE)
        prod3 = kb3 * q3s[...][None]
        s = jnp.sum(prod3, axis=2)
        pos = j * BLK + jax.lax.broadcasted_iota(jnp.int32, (BLK, NH), 0)
        s = jnp.where(pos < ncache, s, -1e30)
        bm = jnp.max(s, axis=0, keepdims=True)
        new_m = jnp.maximum(m[...].T, bm)
        new_mc = new_m.T
        corr = jnp.exp(m[...] - new_mc)
        p = jnp.exp(s - new_m)
        l[...] = l[...] * corr + jnp.sum(p, axis=0, keepdims=True).T
        m[...] = new_mc
        pv3 = jax.lax.broadcast_in_dim(p, (BLK, NH, NOPE), (0, 1))
        acc[...] = acc[...] * corr + jnp.sum(pv3 * vb3, axis=0)

    @pl.when(j == NBLK - 1)
    def _fin():
        o_ref[...] = (acc[...] / l[...]).reshape(1, NH, NOPE)


def _mid_body(ao_ref, x_ref, wo_ref, wn_ref, wg_ref, wse_ref, wsd_ref,
              h2_ref, wt_ref, base_ref):
    attn_res = jax.lax.dot_general(ao_ref[...], wo_ref[...], _DN,
                                   preferred_element_type=_F32)
    resid = x_ref[...] + attn_res
    h2 = _rms(resid, wn_ref[...])
    h2_ref[...] = h2
    wt_ref[...] = jax.lax.dot_general(wg_ref[...], h2, _DN,
                                      preferred_element_type=_F32)
    su = jax.lax.dot_general(h2, wse_ref[...], _DN,
                             preferred_element_type=_F32)
    sg = su[:, :INTER]
    uu = su[:, INTER:]
    act = sg * jax.nn.sigmoid(sg) * uu
    shared = jax.lax.dot_general(act, wsd_ref[...], _DN,
                                 preferred_element_type=_F32)
    base_ref[...] = resid + shared


def _route_sc_body(lg_hbm, out_hbm, lg_v, wt_v):
    cid = jax.lax.axis_index("c")
    sid = jax.lax.axis_index("s")

    @pl.when((cid == 0) & (sid == 0))
    def _():
        pltpu.sync_copy(lg_hbm, lg_v)
        rows = [lg_v[e, :] for e in range(E)]
        m1 = rows[0]
        for e in range(1, E):
            m1 = jnp.maximum(m1, rows[e])
        i1 = jnp.full((B,), E, jnp.int32)
        for e in range(E - 1, -1, -1):
            i1 = jnp.where(rows[e] == m1, jnp.int32(e), i1)
        neg = jnp.full((B,), -jnp.inf, jnp.float32)
        m2 = neg
        for e in range(E):
            m2 = jnp.maximum(m2, jnp.where(i1 == e, neg, rows[e]))
        i2 = jnp.full((B,), E, jnp.int32)
        for e in range(E - 1, -1, -1):
            v = jnp.where(i1 == e, neg, rows[e])
            i2 = jnp.where(v == m2, jnp.int32(e), i2)
        e2 = jnp.exp(m2 - m1)
        d = 1.0 + e2
        wa = 1.0 / d
        wb = e2 / d
        z = jnp.zeros((B,), jnp.float32)
        for e in range(E):
            wt_v[e, :] = jnp.where(i1 == e, wa, z) + jnp.where(i2 == e, wb, z)
        pltpu.sync_copy(wt_v, out_hbm)


def _route_sc(lgt):
    return pl.kernel(
        _route_sc_body,
        out_type=jax.ShapeDtypeStruct((E, B), jnp.float32),
        mesh=plsc.VectorSubcoreMesh(core_axis_name="c", subcore_axis_name="s"),
        scratch_types=[pltpu.VMEM((E, B), jnp.float32),
                       pltpu.VMEM((E, B), jnp.float32)],
    )(lgt)


def _moe_body(x_ref, base_ref, wta_ref, wtb_ref, w1ga_ref, w1ua_ref,
              w2a_ref, w1gb_ref, w1ub_ref, w2b_ref, o_ref):
    p = pl.program_id(0)
    c = pl.program_id(1)

    @pl.when((p == 0) & (c == 0))
    def _seed():
        o_ref[...] = base_ref[...]

    x = x_ref[...]

    def _contrib(w1g_ref, w1u_ref, w2_ref, wt_ref):
        g = jax.lax.dot_general(x, w1g_ref[...].reshape(CHUNK, HID), _DN,
                                preferred_element_type=_F32)
        u = jax.lax.dot_general(x, w1u_ref[...].reshape(CHUNK, HID), _DN,
                                preferred_element_type=_F32)
        act = g * jax.nn.sigmoid(g) * u * wt_ref[...].reshape(B, 1)
        return jax.lax.dot_general(act, w2_ref[...].reshape(HID, CHUNK),
                                   _DN, preferred_element_type=_F32)

    o_ref[...] += (_contrib(w1ga_ref, w1ua_ref, w2a_ref, wta_ref)
                   + _contrib(w1gb_ref, w1ub_ref, w2b_ref, wtb_ref))


def kernel(hidden_states, positions, kv_cache, seq_lens, slot_mapping,
           ln1_w, ln2_w, Wq, Wkv, Wo, Wg, w1, w2, Wse, Wsd):
    x = hidden_states[:, 0, :]
    sl = jnp.maximum(seq_lens, 1).astype(jnp.int32)
    Wq3 = Wq.reshape(NH, QD, HID)

    q_nope, k_cur, v_cur = pl.pallas_call(
        _pre_body,
        grid=(1,),
        in_specs=[
            pl.BlockSpec((B, HID), lambda i: (0, 0)),
            pl.BlockSpec((1, HID), lambda i: (0, 0)),
            pl.BlockSpec((NH, NOPE, HID), lambda i: (0, 0, 0)),
            pl.BlockSpec((2 * HD, HID), lambda i: (0, 0)),
        ],
        out_specs=[pl.BlockSpec((B, HD), lambda i: (0, 0))] * 3,
        out_shape=[jax.ShapeDtypeStruct((B, HD), _F32),
                   jax.ShapeDtypeStruct((B, HD), _F32),
                   jax.ShapeDtypeStruct((B, HD), _F32)],
    )(x, ln1_w.reshape(1, HID), Wq3, Wkv)

    q_nope = q_nope.reshape(B, NH, NOPE)
    k_cur = k_cur.reshape(B, NH, NOPE)
    v_cur = v_cur.reshape(B, NH, NOPE)

    kvm = kv_cache.reshape(2, B, KV * NH, NOPE)

    def _kv_idx(part):
        def idx(b, j, sl_ref):
            nb = (sl_ref[b] - 1 + BLK - 1) // BLK
            return (part, b, jnp.minimum(j, jnp.maximum(nb - 1, 0)), 0)
        return idx

    attn_out = pl.pallas_call(
        _attn_body,
        grid_spec=pltpu.PrefetchScalarGridSpec(
            num_scalar_prefetch=1,
            grid=(B, NBLK),
            in_specs=[
                pl.BlockSpec((1, NH, NOPE), lambda b, j, s: (b, 0, 0)),
                pl.BlockSpec((1, NH, NOPE), lambda b, j, s: (b, 0, 0)),
                pl.BlockSpec((1, NH, NOPE), lambda b, j, s: (b, 0, 0)),
                pl.BlockSpec((1, 1, BLK * NH, NOPE), _kv_idx(0)),
                pl.BlockSpec((1, 1, BLK * NH, NOPE), _kv_idx(1)),
            ],
            out_specs=pl.BlockSpec((1, NH, NOPE), lambda b, j, s: (b, 0, 0)),
            scratch_shapes=[pltpu.VMEM((NH, NOPE), _F32),
                            pltpu.VMEM((NH, NOPE), _F32),
                            pltpu.VMEM((NH, 1), _F32),
                            pltpu.VMEM((NH, 1), _F32)],
        ),
        out_shape=jax.ShapeDtypeStruct((B, NH, NOPE), _F32),
        compiler_params=pltpu.CompilerParams(
            dimension_semantics=("arbitrary", "arbitrary")),
    )(sl, q_nope, k_cur, v_cur, kvm, kvm)

    h2, lgt, base = pl.pallas_call(
        _mid_body,
        out_shape=[jax.ShapeDtypeStruct((B, HID), _F32),
                   jax.ShapeDtypeStruct((E, B), _F32),
                   jax.ShapeDtypeStruct((B, HID), _F32)],
    )(attn_out.reshape(B, HD), x, Wo, ln2_w.reshape(1, HID), Wg, Wse, Wsd)

    wt3 = _route_sc(lgt)[:, :, None]

    out = pl.pallas_call(
        _moe_body,
        grid=(E2, NC),
        in_specs=[
            pl.BlockSpec((B, HID), lambda p, c: (0, 0)),
            pl.BlockSpec((B, HID), lambda p, c: (0, 0)),
            pl.BlockSpec((1, B, 1), lambda p, c: (2 * p, 0, 0)),
            pl.BlockSpec((1, B, 1), lambda p, c: (2 * p + 1, 0, 0)),
            pl.BlockSpec((1, CHUNK, HID), lambda p, c: (2 * p, c, 0)),
            pl.BlockSpec((1, CHUNK, HID), lambda p, c: (2 * p, NC + c, 0)),
            pl.BlockSpec((1, HID, CHUNK), lambda p, c: (2 * p, 0, c)),
            pl.BlockSpec((1, CHUNK, HID), lambda p, c: (2 * p + 1, c, 0)),
            pl.BlockSpec((1, CHUNK, HID),
                         lambda p, c: (2 * p + 1, NC + c, 0)),
            pl.BlockSpec((1, HID, CHUNK), lambda p, c: (2 * p + 1, 0, c)),
        ],
        out_specs=pl.BlockSpec((B, HID), lambda p, c: (0, 0)),
        out_shape=jax.ShapeDtypeStruct((B, HID), _F32),
    )(h2, base, wt3, wt3, w1, w1, w2, w1, w1, w2)

    return out[:, None, :]


# speedup vs baseline: 1.5897x; 1.0039x over previous
import functools

import jax
import jax.numpy as jnp
from jax.experimental import pallas as pl
from jax.experimental.pallas import tpu as pltpu
from jax.experimental.pallas import tpu_sc as plsc

B = 16
HID = 2048
NH = 16
NOPE = 64
ROPE = 32
VD = 64
QD = NOPE + ROPE
E = 8
INTER = 1408
KV = 2048
EPS = 1e-06
SCALE = QD ** -0.5

HD = NH * NOPE
BLK = 256
NBLK = KV // BLK
CHUNK = 128
NC = INTER // CHUNK
E2 = E // 2

_F32 = jnp.float32
_DN = (((1,), (1,)), ((), ()))
_DN0 = (((1,), (0,)), ((), ()))


def _rms(x, w):
    return x * jax.lax.rsqrt(jnp.mean(x * x, axis=-1, keepdims=True) + EPS) * w


PRE_C = 4
QC = HD // PRE_C
KC = HD // PRE_C


def _pre_body(x_ref, w_ref, wq_ref, wkc_ref, wvc_ref, qn_ref, kc_ref, vc_ref):
    xn = _rms(x_ref[...], w_ref[...])
    wq = wq_ref[...].reshape(QC, HID)
    qn_ref[...] = jax.lax.dot_general(xn, wq, _DN,
                                      preferred_element_type=_F32) * SCALE
    kc_ref[...] = jax.lax.dot_general(xn, wkc_ref[...], _DN,
                                      preferred_element_type=_F32)
    vc_ref[...] = jax.lax.dot_general(xn, wvc_ref[...], _DN,
                                      preferred_element_type=_F32)


def _attn_body(sl_ref, q_ref, kc_ref, vc_ref, kb_ref, vb_ref,
               o_ref, q3s, acc, m, l):
    b = pl.program_id(0)
    j = pl.program_id(1)
    ncache = sl_ref[b] - 1
    nb = (ncache + BLK - 1) // BLK

    @pl.when(j == 0)
    def _init():
        q3 = q_ref[...].reshape(NH, NOPE)
        q3s[...] = q3
        kc3 = kc_ref[...].reshape(NH, NOPE)
        s_cur = jnp.sum(q3 * kc3, axis=1, keepdims=True)
        m[...] = s_cur
        l[...] = jnp.ones_like(s_cur)
        acc[...] = vc_ref[...].reshape(NH, NOPE)

    @pl.when(j < nb)
    def _block():
        kb3 = kb_ref[...].reshape(BLK, NH, NOPE)
        vb3 = vb_ref[...].reshape(BLK, NH, NOPE)
        prod3 = kb3 * q3s[...][None]
        s = jnp.sum(prod3, axis=2)
        pos = j * BLK + jax.lax.broadcasted_iota(jnp.int32, (BLK, NH), 0)
        s = jnp.where(pos < ncache, s, -1e30)
        bm = jnp.max(s, axis=0, keepdims=True)
        new_m = jnp.maximum(m[...].T, bm)
        new_mc = new_m.T
        corr = jnp.exp(m[...] - new_mc)
        p = jnp.exp(s - new_m)
        l[...] = l[...] * corr + jnp.sum(p, axis=0, keepdims=True).T
        m[...] = new_mc
        pv3 = jax.lax.broadcast_in_dim(p, (BLK, NH, NOPE), (0, 1))
        acc[...] = acc[...] * corr + jnp.sum(pv3 * vb3, axis=0)

    @pl.when(j == NBLK - 1)
    def _fin():
        o_ref[...] = (acc[...] / l[...]).reshape(1, NH, NOPE)


def _mid_body(ao_ref, x_ref, wo_ref, wn_ref, wg_ref, wse_ref, wsd_ref,
              h2_ref, wt_ref, base_ref):
    attn_res = jax.lax.dot_general(ao_ref[...], wo_ref[...], _DN,
                                   preferred_element_type=_F32)
    resid = x_ref[...] + attn_res
    h2 = _rms(resid, wn_ref[...])
    h2_ref[...] = h2
    wt_ref[...] = jax.lax.dot_general(wg_ref[...], h2, _DN,
                                      preferred_element_type=_F32)
    su = jax.lax.dot_general(h2, wse_ref[...], _DN,
                             preferred_element_type=_F32)
    sg = su[:, :INTER]
    uu = su[:, INTER:]
    act = sg * jax.nn.sigmoid(sg) * uu
    shared = jax.lax.dot_general(act, wsd_ref[...], _DN,
                                 preferred_element_type=_F32)
    base_ref[...] = resid + shared


def _route_sc_body(lg_hbm, out_hbm, lg_v, wt_v):
    cid = jax.lax.axis_index("c")
    sid = jax.lax.axis_index("s")

    @pl.when((cid == 0) & (sid == 0))
    def _():
        pltpu.sync_copy(lg_hbm, lg_v)
        rows = [lg_v[e, :] for e in range(E)]
        m1 = rows[0]
        for e in range(1, E):
            m1 = jnp.maximum(m1, rows[e])
        i1 = jnp.full((B,), E, jnp.int32)
        for e in range(E - 1, -1, -1):
            i1 = jnp.where(rows[e] == m1, jnp.int32(e), i1)
        neg = jnp.full((B,), -jnp.inf, jnp.float32)
        m2 = neg
        for e in range(E):
            m2 = jnp.maximum(m2, jnp.where(i1 == e, neg, rows[e]))
        i2 = jnp.full((B,), E, jnp.int32)
        for e in range(E - 1, -1, -1):
            v = jnp.where(i1 == e, neg, rows[e])
            i2 = jnp.where(v == m2, jnp.int32(e), i2)
        e2 = jnp.exp(m2 - m1)
        d = 1.0 + e2
        wa = 1.0 / d
        wb = e2 / d
        z = jnp.zeros((B,), jnp.float32)
        for e in range(E):
            wt_v[e, :] = jnp.where(i1 == e, wa, z) + jnp.where(i2 == e, wb, z)
        pltpu.sync_copy(wt_v, out_hbm)


def _route_sc(lgt):
    return pl.kernel(
        _route_sc_body,
        out_type=jax.ShapeDtypeStruct((E, B), jnp.float32),
        mesh=plsc.VectorSubcoreMesh(core_axis_name="c", subcore_axis_name="s"),
        scratch_types=[pltpu.VMEM((E, B), jnp.float32),
                       pltpu.VMEM((E, B), jnp.float32)],
    )(lgt)


def _moe_body(x_ref, base_ref, wta_ref, wtb_ref, w1ga_ref, w1ua_ref,
              w2a_ref, w1gb_ref, w1ub_ref, w2b_ref, o_ref):
    p = pl.program_id(0)
    c = pl.program_id(1)

    @pl.when((p == 0) & (c == 0))
    def _seed():
        o_ref[...] = base_ref[...]

    x = x_ref[...]

    def _contrib(w1g_ref, w1u_ref, w2_ref, wt_ref):
        g = jax.lax.dot_general(x, w1g_ref[...].reshape(CHUNK, HID), _DN,
                                preferred_element_type=_F32)
        u = jax.lax.dot_general(x, w1u_ref[...].reshape(CHUNK, HID), _DN,
                                preferred_element_type=_F32)
        act = g * jax.nn.sigmoid(g) * u * wt_ref[...].reshape(B, 1)
        return jax.lax.dot_general(act, w2_ref[...].reshape(HID, CHUNK),
                                   _DN, preferred_element_type=_F32)

    o_ref[...] += (_contrib(w1ga_ref, w1ua_ref, w2a_ref, wta_ref)
                   + _contrib(w1gb_ref, w1ub_ref, w2b_ref, wtb_ref))


def kernel(hidden_states, positions, kv_cache, seq_lens, slot_mapping,
           ln1_w, ln2_w, Wq, Wkv, Wo, Wg, w1, w2, Wse, Wsd):
    x = hidden_states[:, 0, :]
    sl = jnp.maximum(seq_lens, 1).astype(jnp.int32)
    Wq3 = Wq.reshape(NH, QD, HID)

    q_nope, k_cur, v_cur = pl.pallas_call(
        _pre_body,
        grid=(PRE_C,),
        in_specs=[
            pl.BlockSpec((B, HID), lambda c: (0, 0)),
            pl.BlockSpec((1, HID), lambda c: (0, 0)),
            pl.BlockSpec((NH // PRE_C, NOPE, HID), lambda c: (c, 0, 0)),
            pl.BlockSpec((KC, HID), lambda c: (c, 0)),
            pl.BlockSpec((KC, HID), lambda c: (PRE_C + c, 0)),
        ],
        out_specs=[pl.BlockSpec((B, QC), lambda c: (0, c)),
                   pl.BlockSpec((B, KC), lambda c: (0, c)),
                   pl.BlockSpec((B, KC), lambda c: (0, c))],
        out_shape=[jax.ShapeDtypeStruct((B, HD), _F32),
                   jax.ShapeDtypeStruct((B, HD), _F32),
                   jax.ShapeDtypeStruct((B, HD), _F32)],
    )(x, ln1_w.reshape(1, HID), Wq3, Wkv, Wkv)

    q_nope = q_nope.reshape(B, NH, NOPE)
    k_cur = k_cur.reshape(B, NH, NOPE)
    v_cur = v_cur.reshape(B, NH, NOPE)

    kvm = kv_cache.reshape(2, B, KV * NH, NOPE)

    def _kv_idx(part):
        def idx(b, j, sl_ref):
            nb = (sl_ref[b] - 1 + BLK - 1) // BLK
            return (part, b, jnp.minimum(j, jnp.maximum(nb - 1, 0)), 0)
        return idx

    attn_out = pl.pallas_call(
        _attn_body,
        grid_spec=pltpu.PrefetchScalarGridSpec(
            num_scalar_prefetch=1,
            grid=(B, NBLK),
            in_specs=[
                pl.BlockSpec((1, NH, NOPE), lambda b, j, s: (b, 0, 0)),
                pl.BlockSpec((1, NH, NOPE), lambda b, j, s: (b, 0, 0)),
                pl.BlockSpec((1, NH, NOPE), lambda b, j, s: (b, 0, 0)),
                pl.BlockSpec((1, 1, BLK * NH, NOPE), _kv_idx(0)),
                pl.BlockSpec((1, 1, BLK * NH, NOPE), _kv_idx(1)),
            ],
            out_specs=pl.BlockSpec((1, NH, NOPE), lambda b, j, s: (b, 0, 0)),
            scratch_shapes=[pltpu.VMEM((NH, NOPE), _F32),
                            pltpu.VMEM((NH, NOPE), _F32),
                            pltpu.VMEM((NH, 1), _F32),
                            pltpu.VMEM((NH, 1), _F32)],
        ),
        out_shape=jax.ShapeDtypeStruct((B, NH, NOPE), _F32),
        compiler_params=pltpu.CompilerParams(
            dimension_semantics=("arbitrary", "arbitrary")),
    )(sl, q_nope, k_cur, v_cur, kvm, kvm)

    h2, lgt, base = pl.pallas_call(
        _mid_body,
        out_shape=[jax.ShapeDtypeStruct((B, HID), _F32),
                   jax.ShapeDtypeStruct((E, B), _F32),
                   jax.ShapeDtypeStruct((B, HID), _F32)],
    )(attn_out.reshape(B, HD), x, Wo, ln2_w.reshape(1, HID), Wg, Wse, Wsd)

    wt3 = _route_sc(lgt)[:, :, None]

    out = pl.pallas_call(
        _moe_body,
        grid=(E2, NC),
        in_specs=[
            pl.BlockSpec((B, HID), lambda p, c: (0, 0)),
            pl.BlockSpec((B, HID), lambda p, c: (0, 0)),
            pl.BlockSpec((1, B, 1), lambda p, c: (2 * p, 0, 0)),
            pl.BlockSpec((1, B, 1), lambda p, c: (2 * p + 1, 0, 0)),
            pl.BlockSpec((1, CHUNK, HID), lambda p, c: (2 * p, c, 0)),
            pl.BlockSpec((1, CHUNK, HID), lambda p, c: (2 * p, NC + c, 0)),
            pl.BlockSpec((1, HID, CHUNK), lambda p, c: (2 * p, 0, c)),
            pl.BlockSpec((1, CHUNK, HID), lambda p, c: (2 * p + 1, c, 0)),
            pl.BlockSpec((1, CHUNK, HID),
                         lambda p, c: (2 * p + 1, NC + c, 0)),
            pl.BlockSpec((1, HID, CHUNK), lambda p, c: (2 * p + 1, 0, c)),
        ],
        out_specs=pl.BlockSpec((B, HID), lambda p, c: (0, 0)),
        out_shape=jax.ShapeDtypeStruct((B, HID), _F32),
    )(h2, base, wt3, wt3, w1, w1, w2, w1, w1, w2)

    return out[:, None, :]
